# baseline (speedup 1.0000x reference)
import dataclasses

import jax
import jax.numpy as jnp
from jax import lax
from jax.experimental import pallas as pl
from jax.experimental.pallas import tpu as pltpu
from jax.experimental.pallas import tpu_sc as plsc

N_NODES = 10000
N_EDGES = 320000
HID = 128
NW = 32
LANES = 16
NPAD = 10240
NPT = NPAD // 16
ZB = 8
EPAD = 327680
EPW = EPAD // NW

CBA = 128
NCA = EPW // CBA
NCI = 16
NSUP = NCA // NCI
CBE = 64
NCE = EPW // CBE


def _zero_fill(buf, rows, width):
    @pl.loop(0, rows)
    def _r(r):
        for j in range(width // LANES):
            buf[r, pl.ds(j * LANES, LANES)] = jnp.zeros((LANES,), jnp.float32)


def _mesh():
    return plsc.VectorSubcoreMesh(core_axis_name="c", subcore_axis_name="s")


def _wid():
    return lax.axis_index("s") * 2 + lax.axis_index("c")


def _deg_body(dst_hbm, out_hbm, idx2, cnt_v, sem):
    wid = _wid()

    @pl.loop(0, NPAD // LANES)
    def _z(r):
        cnt_v[pl.ds(r * LANES, LANES)] = jnp.zeros((LANES,), jnp.float32)

    pltpu.sync_copy(dst_hbm.at[wid], idx2)
    ones = jnp.ones((LANES,), jnp.float32)

    @pl.loop(0, NCA)
    def _chunk(i):
        @pl.loop(0, CBA // LANES)
        def _g(g):
            idx = idx2[i, pl.ds(g * LANES, LANES)]
            plsc.addupdate_scatter(cnt_v, [idx], ones)

    pltpu.sync_copy(cnt_v, out_hbm.at[wid])


@jax.jit
def _deg_kernel(dst3):
    cp = pltpu.CompilerParams()
    if "needs_layout_passes" in pltpu.CompilerParams.__dataclass_fields__:
        cp = dataclasses.replace(cp, needs_layout_passes=False)
    k = pl.kernel(
        _deg_body,
        out_type=jax.ShapeDtypeStruct((NW, NPAD), jnp.float32),
        mesh=_mesh(),
        compiler_params=cp,
        scratch_types=[
            pltpu.VMEM((NCA, CBA), jnp.int32),
            pltpu.VMEM((NPAD,), jnp.float32),
            pltpu.SemaphoreType.DMA,
        ],
    )
    return k(dst3)


def _agg_body(u_hbm, src_hbm, dst_hbm, out_hbm,
              sidxs, didxs, rows0, rows1, zero_v, acc_sh,
              gsem0, gsem1, ssem0, ssem1):
    cid = lax.axis_index("c")
    sid = lax.axis_index("s")
    wid = sid * 2 + cid
    rows = (rows0, rows1)
    gsem = (gsem0, gsem1)
    ssem = (ssem0, ssem1)

    _zero_fill(zero_v, ZB, HID)
    for t in range(NPT // ZB):
        pltpu.sync_copy(zero_v, acc_sh.at[pl.ds(sid * NPT + t * ZB, ZB)])
    plsc.subcore_barrier()

    @pl.loop(0, NSUP)
    def _sup(sc):
        pltpu.sync_copy(src_hbm.at[wid, pl.ds(sc * NCI, NCI)], sidxs)
        pltpu.sync_copy(dst_hbm.at[wid, pl.ds(sc * NCI, NCI)], didxs)
        pltpu.async_copy(u_hbm.at[sidxs.at[0]], rows0, gsem0)

        @pl.loop(0, NCI, step=2)
        def _chunk(i):
            for b in range(2):
                cur = i + b
                pltpu.make_async_copy(u_hbm.at[sidxs.at[cur]], rows[b],
                                      gsem[b]).wait()
                pltpu.async_copy(rows[b], acc_sh.at[didxs.at[cur]],
                                 ssem[b], add=True)
                @pl.when(cur + 1 < NCI)
                def _refill():
                    @pl.when(cur >= 1)
                    def _drain():
                        pltpu.make_async_copy(rows[1 - b],
                                              acc_sh.at[didxs.at[cur - 1]],
                                              ssem[1 - b]).wait()
                    pltpu.async_copy(u_hbm.at[sidxs.at[cur + 1]], rows[1 - b],
                                     gsem[1 - b])

        pltpu.make_async_copy(rows[0], acc_sh.at[didxs.at[NCI - 2]],
                              ssem0).wait()
        pltpu.make_async_copy(rows[1], acc_sh.at[didxs.at[NCI - 1]],
                              ssem1).wait()

    plsc.subcore_barrier()
    row0 = cid * NPAD + sid * NPT
    pltpu.sync_copy(acc_sh.at[pl.ds(sid * NPT, NPT)],
                    out_hbm.at[pl.ds(row0, NPT)])


@jax.jit
def _agg_kernel(u, src3, dst3):
    k = pl.kernel(
        _agg_body,
        out_type=jax.ShapeDtypeStruct((2 * NPAD, HID), jnp.float32),
        mesh=_mesh(),
        scratch_types=[
            pltpu.VMEM((NCI, CBA), jnp.int32),
            pltpu.VMEM((NCI, CBA), jnp.int32),
            pltpu.VMEM((CBA, HID), jnp.float32),
            pltpu.VMEM((CBA, HID), jnp.float32),
            pltpu.VMEM((ZB, HID), jnp.float32),
            pltpu.VMEM_SHARED((NPAD, HID), jnp.float32),
            pltpu.SemaphoreType.DMA,
            pltpu.SemaphoreType.DMA,
            pltpu.SemaphoreType.DMA,
            pltpu.SemaphoreType.DMA,
        ],
    )
    return k(u, src3, dst3)


def _edge_stage_body(a_hbm, b_hbm, src_hbm, dst_hbm, c_hbm, w_hbm, out_hbm,
                     sidx2, didx2,
                     ra0, ra1, rb0, rb1, cv0, cv1, w_v, ov0, ov1,
                     isem0, isem1, osem0, osem1):
    wid = _wid()
    ra = (ra0, ra1)
    rb = (rb0, rb1)
    cv = (cv0, cv1)
    ov = (ov0, ov1)
    isem = (isem0, isem1)
    osem = (osem0, osem1)
    base = wid * EPW
    pltpu.sync_copy(w_hbm, w_v)
    pltpu.sync_copy(src_hbm.at[wid], sidx2)
    pltpu.sync_copy(dst_hbm.at[wid], didx2)

    def issue(cur, b):
        pltpu.async_copy(a_hbm.at[sidx2.at[cur]], ra[b], isem[b])
        pltpu.async_copy(b_hbm.at[didx2.at[cur]], rb[b], isem[b])
        pltpu.async_copy(c_hbm.at[pl.ds(base + cur * CBE, CBE)], cv[b],
                         isem[b])

    def wait_in(cur, b):
        pltpu.make_async_copy(a_hbm.at[sidx2.at[cur]], ra[b], isem[b]).wait()
        pltpu.make_async_copy(b_hbm.at[didx2.at[cur]], rb[b], isem[b]).wait()
        pltpu.make_async_copy(c_hbm.at[pl.ds(base + cur * CBE, CBE)], cv[b],
                              isem[b]).wait()

    issue(0, 0)

    @pl.loop(0, NCE, step=2)
    def _chunk(i):
        for b in range(2):
            cur = i + b
            wait_in(cur, b)

            @pl.when(cur + 1 < NCE)
            def _refill():
                @pl.when(cur >= 1)
                def _drain():
                    pltpu.make_async_copy(
                        ov[1 - b],
                        out_hbm.at[pl.ds(base + (cur - 1) * CBE, CBE)],
                        osem[1 - b]).wait()
                issue(cur + 1, 1 - b)

            @pl.loop(0, CBE)
            def _edge(e):
                acc = jnp.zeros((LANES,), jnp.float32)
                for j in range(HID // LANES):
                    sl = pl.ds(j * LANES, LANES)
                    g = ra[b][e, sl] + rb[b][e, sl] + cv[b][e, sl]
                    g = jnp.maximum(g, 0.0)
                    acc = acc + g * w_v[sl]
                ov[b][e, :] = acc

            pltpu.async_copy(ov[b], out_hbm.at[pl.ds(base + cur * CBE, CBE)],
                             osem[b])

    pltpu.make_async_copy(ov0, out_hbm.at[pl.ds(base + (NCE - 2) * CBE, CBE)],
                          osem0).wait()
    pltpu.make_async_copy(ov1, out_hbm.at[pl.ds(base + (NCE - 1) * CBE, CBE)],
                          osem1).wait()


@jax.jit
def _edge_stage(A, B, src3, dst3, c, w):
    k = pl.kernel(
        _edge_stage_body,
        out_type=jax.ShapeDtypeStruct((EPAD, LANES), jnp.float32),
        mesh=_mesh(),
        scratch_types=[
            pltpu.VMEM((NCE, CBE), jnp.int32),
            pltpu.VMEM((NCE, CBE), jnp.int32),
            pltpu.VMEM((CBE, HID), jnp.float32),
            pltpu.VMEM((CBE, HID), jnp.float32),
            pltpu.VMEM((CBE, HID), jnp.float32),
            pltpu.VMEM((CBE, HID), jnp.float32),
            pltpu.VMEM((CBE, HID), jnp.float32),
            pltpu.VMEM((CBE, HID), jnp.float32),
            pltpu.VMEM((HID,), jnp.float32),
            pltpu.VMEM((CBE, LANES), jnp.float32),
            pltpu.VMEM((CBE, LANES), jnp.float32),
            pltpu.SemaphoreType.DMA,
            pltpu.SemaphoreType.DMA,
            pltpu.SemaphoreType.DMA,
            pltpu.SemaphoreType.DMA,
        ],
    )
    return k(A, B, src3, dst3, c, w)


def kernel(x, eIndex, eAttributes, W1, b1, W2, b2, Wm1, bm1, Wm2, bm2):
    src = eIndex[0].astype(jnp.int32)
    dst = eIndex[1].astype(jnp.int32)
    H = W1.shape[1]

    pad = EPAD - N_EDGES
    srcp = jnp.concatenate(
        [src, jnp.full((pad,), N_NODES, jnp.int32)])
    dstp = jnp.concatenate(
        [dst, jnp.full((pad,), N_NODES, jnp.int32)])
    srcA = srcp.reshape(NW, NCA, CBA)
    dstA = dstp.reshape(NW, NCA, CBA)
    srcE = srcp.reshape(NW, NCE, CBE)
    dstE = dstp.reshape(NW, NCE, CBE)

    degp = _deg_kernel(dstA)
    deg = jnp.sum(degp, axis=0)[:N_NODES] + 1.0
    dis = jax.lax.rsqrt(deg)

    def pad_nodes(m):
        return jnp.concatenate(
            [m, jnp.zeros((NPAD - N_NODES, m.shape[1]), m.dtype)])

    u = (x @ W1) * dis[:, None]
    aggp = _agg_kernel(pad_nodes(u), srcA, dstA)
    agg = aggp[:N_NODES] + aggp[NPAD:NPAD + N_NODES]
    h = jax.nn.relu((agg + u) * dis[:, None] + b1)
    u = (h @ W2) * dis[:, None]
    aggp = _agg_kernel(pad_nodes(u), srcA, dstA)
    agg = aggp[:N_NODES] + aggp[NPAD:NPAD + N_NODES]
    h = (agg + u) * dis[:, None] + b2

    A = h @ Wm1[:H]
    B = h @ Wm1[H:2 * H]
    eAp = jnp.concatenate(
        [eAttributes, jnp.zeros((pad, eAttributes.shape[1]), jnp.float32)])
    c = eAp @ Wm1[2 * H:] + bm1
    part = _edge_stage(pad_nodes(A), pad_nodes(B), srcE, dstE, c, Wm2[:, 0])
    return jnp.sum(part[:N_EDGES], axis=1, keepdims=True) + bm2
